# Initial kernel scaffold; baseline (speedup 1.0000x reference)
import jax
import jax.numpy as jnp
from jax.experimental import pallas as pl


def kernel(x, poly_conn, poly_index, qkv_weight, qkv_bias, E_weight, E_bias, conn_lin_weight, conn_lin_bias):
    raise NotImplementedError("write your pallas kernel here")


# speedup vs baseline: 3.8373x; 3.8373x over previous
import functools

import jax
import jax.numpy as jnp
from jax import lax
from jax.experimental import pallas as pl
from jax.experimental.pallas import tpu as pltpu
from jax.experimental.pallas import tpu_sc as plsc

N_NODES = 10000
N_EDGES = 320000
HIDDEN = 128
A = 128

_INFO = plsc.get_sparse_core_info()
_NC = _INFO.num_cores
_NS = _INFO.num_subcores
_NW = _NC * _NS
_EPW = N_EDGES // _NW
_C = 80
_NCHUNK = _EPW // _C
_RPT = 624
_REM_BASE = _NS * _RPT
_ZR = 24


def _fill_zeros(zv):
    z16 = jnp.zeros((16,), jnp.float32)
    for r in range(_ZR):
        for j in range(HIDDEN // 16):
            zv[r, pl.ds(j * 16, 16)] = z16


def _zero_shared(zv, sh, sid):
    for i in range(_RPT // _ZR):
        off = pl.multiple_of(sid * _RPT + i * _ZR, 8)
        pltpu.sync_copy(zv, sh.at[pl.ds(off, _ZR)])

    @pl.when(sid < 2)
    def _():
        off = pl.multiple_of(_REM_BASE + sid * 8, 8)
        pltpu.sync_copy(zv.at[pl.ds(0, 8)], sh.at[pl.ds(off, 8)])


def _copy_out_partial(sh, out_h, cid, sid):
    base = pl.multiple_of(sid * _RPT, 8)
    pltpu.sync_copy(sh.at[pl.ds(base, _RPT)], out_h.at[cid, pl.ds(base, _RPT)])

    @pl.when(sid < 2)
    def _():
        off = pl.multiple_of(_REM_BASE + sid * 8, 8)
        pltpu.sync_copy(sh.at[pl.ds(off, 8)], out_h.at[cid, pl.ds(off, 8)])


def _sc_edge_gather(qh, kh, vh, dst, src):
    mesh = plsc.VectorSubcoreMesh(core_axis_name="c", subcore_axis_name="s")

    @functools.partial(
        pl.kernel,
        out_type=(
            jax.ShapeDtypeStruct((N_EDGES, HIDDEN), jnp.float32),
            jax.ShapeDtypeStruct((N_EDGES, HIDDEN), jnp.float32),
            jax.ShapeDtypeStruct((_NC, N_NODES, HIDDEN), jnp.float32),
        ),
        mesh=mesh,
        scratch_types=(
            pltpu.VMEM((_C,), jnp.int32),
            pltpu.VMEM((_C,), jnp.int32),
            pltpu.VMEM((_C, HIDDEN), jnp.float32),
            pltpu.VMEM((_C, HIDDEN), jnp.float32),
            pltpu.VMEM((_C, HIDDEN), jnp.float32),
            pltpu.VMEM((_ZR, HIDDEN), jnp.float32),
            pltpu.VMEM_SHARED((N_NODES, HIDDEN), jnp.float32),
            pltpu.SemaphoreType.DMA,
            pltpu.SemaphoreType.DMA,
            pltpu.SemaphoreType.DMA,
        ),
    )
    def k(qh_h, kh_h, vh_h, dst_h, src_h, qg_h, kg_h, aggp_h,
          dst_v, src_v, qv, kv, vv, zv, agg_sh, sem_q, sem_k, sem_v):
        cid = lax.axis_index("c")
        sid = lax.axis_index("s")
        wid = sid * _NC + cid

        _fill_zeros(zv)
        _zero_shared(zv, agg_sh, sid)
        plsc.subcore_barrier()

        def body(i, carry):
            base = pl.multiple_of(wid * _EPW + i * _C, _C)
            pltpu.sync_copy(dst_h.at[pl.ds(base, _C)], dst_v)
            pltpu.sync_copy(src_h.at[pl.ds(base, _C)], src_v)
            cq = pltpu.async_copy(qh_h.at[dst_v], qv, sem_q)
            ck = pltpu.async_copy(kh_h.at[src_v], kv, sem_k)
            cv = pltpu.async_copy(vh_h.at[src_v], vv, sem_v)
            cq.wait()
            ck.wait()
            cv.wait()
            pltpu.sync_copy(qv, qg_h.at[pl.ds(base, _C)])
            pltpu.sync_copy(kv, kg_h.at[pl.ds(base, _C)])
            pltpu.sync_copy(vv, agg_sh.at[dst_v], add=True)
            return carry

        lax.fori_loop(0, _NCHUNK, body, 0)
        plsc.subcore_barrier()
        _copy_out_partial(agg_sh, aggp_h, cid, sid)

    return k(qh, kh, vh, dst, src)


def _sc_segsum(conn, dst):
    mesh = plsc.VectorSubcoreMesh(core_axis_name="c", subcore_axis_name="s")

    @functools.partial(
        pl.kernel,
        out_type=jax.ShapeDtypeStruct((_NC, N_NODES, HIDDEN), jnp.float32),
        mesh=mesh,
        scratch_types=(
            pltpu.VMEM((_C,), jnp.int32),
            pltpu.VMEM((_C, HIDDEN), jnp.float32),
            pltpu.VMEM((_ZR, HIDDEN), jnp.float32),
            pltpu.VMEM_SHARED((N_NODES, HIDDEN), jnp.float32),
        ),
    )
    def k(conn_h, dst_h, eaggp_h, dst_v, cv, zv, e_sh):
        cid = lax.axis_index("c")
        sid = lax.axis_index("s")
        wid = sid * _NC + cid

        _fill_zeros(zv)
        _zero_shared(zv, e_sh, sid)
        plsc.subcore_barrier()

        def body(i, carry):
            base = pl.multiple_of(wid * _EPW + i * _C, _C)
            pltpu.sync_copy(dst_h.at[pl.ds(base, _C)], dst_v)
            pltpu.sync_copy(conn_h.at[pl.ds(base, _C)], cv)
            pltpu.sync_copy(cv, e_sh.at[dst_v], add=True)
            return carry

        lax.fori_loop(0, _NCHUNK, body, 0)
        plsc.subcore_barrier()
        _copy_out_partial(e_sh, eaggp_h, cid, sid)

    return k(conn, dst)


def _tc_qkv(x, qkv_weight, qkv_bias):
    bm = 1000

    def body(x_ref, w_ref, b_ref, q_ref, k_ref, v_ref):
        r = lax.dot_general(
            x_ref[...], w_ref[...], (((1,), (1,)), ((), ())),
            preferred_element_type=jnp.float32,
            precision=lax.Precision.HIGHEST,
        ) + b_ref[...]
        q_ref[...] = r[:, :A]
        k_ref[...] = r[:, A:2 * A]
        v_ref[...] = r[:, 2 * A:]

    return pl.pallas_call(
        body,
        grid=(N_NODES // bm,),
        in_specs=[
            pl.BlockSpec((bm, HIDDEN), lambda i: (i, 0)),
            pl.BlockSpec((3 * A, HIDDEN), lambda i: (0, 0)),
            pl.BlockSpec((1, 3 * A), lambda i: (0, 0)),
        ],
        out_specs=[pl.BlockSpec((bm, HIDDEN), lambda i: (i, 0))] * 3,
        out_shape=[jax.ShapeDtypeStruct((N_NODES, HIDDEN), jnp.float32)] * 3,
    )(x, qkv_weight, qkv_bias)


def _tc_edge(poly_conn, qg, kg, E_weight, E_bias):
    be = 2000

    def body(pc_ref, qg_ref, kg_ref, w_ref, b_ref, conn_ref):
        eh = lax.dot_general(
            pc_ref[...], w_ref[...], (((1,), (1,)), ((), ())),
            preferred_element_type=jnp.float32,
            precision=lax.Precision.HIGHEST,
        ) + b_ref[...]
        m = qg_ref[...] + kg_ref[...]
        c1 = m * eh[:, :A]
        c2 = jnp.sqrt(jnp.maximum(c1, 0.0)) - jnp.sqrt(jnp.maximum(-c1, 0.0))
        conn_ref[...] = jnp.maximum(c2 + eh[:, A:], 0.0)

    return pl.pallas_call(
        body,
        grid=(N_EDGES // be,),
        in_specs=[
            pl.BlockSpec((be, HIDDEN), lambda i: (i, 0)),
            pl.BlockSpec((be, HIDDEN), lambda i: (i, 0)),
            pl.BlockSpec((be, HIDDEN), lambda i: (i, 0)),
            pl.BlockSpec((2 * A, HIDDEN), lambda i: (0, 0)),
            pl.BlockSpec((1, 2 * A), lambda i: (0, 0)),
        ],
        out_specs=pl.BlockSpec((be, HIDDEN), lambda i: (i, 0)),
        out_shape=jax.ShapeDtypeStruct((N_EDGES, HIDDEN), jnp.float32),
    )(poly_conn, qg, kg, E_weight, E_bias)


def _tc_final(a0, a1, e0, e1, w, b):
    bm = 1000

    def body(a0_ref, a1_ref, e0_ref, e1_ref, w_ref, b_ref, o_ref):
        eagg = e0_ref[...] + e1_ref[...]
        o_ref[...] = a0_ref[...] + a1_ref[...] + lax.dot_general(
            eagg, w_ref[...], (((1,), (1,)), ((), ())),
            preferred_element_type=jnp.float32,
            precision=lax.Precision.HIGHEST,
        ) + b_ref[...]

    blk = pl.BlockSpec((bm, HIDDEN), lambda i: (i, 0))
    return pl.pallas_call(
        body,
        grid=(N_NODES // bm,),
        in_specs=[blk, blk, blk, blk,
                  pl.BlockSpec((HIDDEN, A), lambda i: (0, 0)),
                  pl.BlockSpec((1, HIDDEN), lambda i: (0, 0))],
        out_specs=blk,
        out_shape=jax.ShapeDtypeStruct((N_NODES, HIDDEN), jnp.float32),
    )(a0, a1, e0, e1, w, b)


def kernel(x, poly_conn, poly_index, qkv_weight, qkv_bias, E_weight, E_bias,
           conn_lin_weight, conn_lin_bias):
    qh, kh, vh = _tc_qkv(x, qkv_weight, qkv_bias.reshape(1, -1))
    dst = poly_index[0]
    src = poly_index[1]
    qg, kg, aggp = _sc_edge_gather(qh, kh, vh, dst, src)
    conn = _tc_edge(poly_conn, qg, kg, E_weight, E_bias.reshape(1, -1))
    eaggp = _sc_segsum(conn, dst)
    no = _tc_final(aggp[0], aggp[1], eaggp[0], eaggp[1],
                   conn_lin_weight, conn_lin_bias.reshape(1, -1))
    return no, conn


# speedup vs baseline: 5.3759x; 1.4010x over previous
import functools

import jax
import jax.numpy as jnp
from jax import lax
from jax.experimental import pallas as pl
from jax.experimental.pallas import tpu as pltpu
from jax.experimental.pallas import tpu_sc as plsc

N_NODES = 10000
N_EDGES = 320000
HIDDEN = 128
A = 128

_INFO = plsc.get_sparse_core_info()
_NC = _INFO.num_cores
_NS = _INFO.num_subcores
_NW = _NC * _NS
_EPW = N_EDGES // _NW
_C = 80
_NCHUNK = _EPW // _C
_RPT = 624
_REM_BASE = _NS * _RPT


def _init_shared(zeros_h, sh, sid):
    base = pl.multiple_of(sid * _RPT, 8)
    pltpu.sync_copy(zeros_h.at[pl.ds(base, _RPT)], sh.at[pl.ds(base, _RPT)])

    @pl.when(sid < 2)
    def _():
        off = pl.multiple_of(_REM_BASE + sid * 8, 8)
        pltpu.sync_copy(zeros_h.at[pl.ds(off, 8)], sh.at[pl.ds(off, 8)])


def _copy_out_partial(sh, out_h, cid, sid):
    base = pl.multiple_of(sid * _RPT, 8)
    pltpu.sync_copy(sh.at[pl.ds(base, _RPT)], out_h.at[cid, pl.ds(base, _RPT)])

    @pl.when(sid < 2)
    def _():
        off = pl.multiple_of(_REM_BASE + sid * 8, 8)
        pltpu.sync_copy(sh.at[pl.ds(off, 8)], out_h.at[cid, pl.ds(off, 8)])


def _sc_qk_gather(qh, kh, dst3, src3):
    mesh = plsc.VectorSubcoreMesh(core_axis_name="c", subcore_axis_name="s")

    @functools.partial(
        pl.kernel,
        out_type=(
            jax.ShapeDtypeStruct((N_EDGES, HIDDEN), jnp.float32),
            jax.ShapeDtypeStruct((N_EDGES, HIDDEN), jnp.float32),
        ),
        mesh=mesh,
        scratch_types=(
            pltpu.VMEM((_NCHUNK, _C), jnp.int32),
            pltpu.VMEM((_NCHUNK, _C), jnp.int32),
            pltpu.VMEM((2, _C, HIDDEN), jnp.float32),
            pltpu.VMEM((2, _C, HIDDEN), jnp.float32),
            pltpu.SemaphoreType.DMA,
            pltpu.SemaphoreType.DMA,
            pltpu.SemaphoreType.DMA,
            pltpu.SemaphoreType.DMA,
        ),
    )
    def k(qh_h, kh_h, dst_h, src_h, qg_h, kg_h,
          dst_v, src_v, qv, kv, sem_g0, sem_g1, sem_w0, sem_w1):
        cid = lax.axis_index("c")
        sid = lax.axis_index("s")
        wid = sid * _NC + cid
        sem_g = (sem_g0, sem_g1)
        sem_w = (sem_w0, sem_w1)

        pltpu.sync_copy(dst_h.at[wid], dst_v)
        pltpu.sync_copy(src_h.at[wid], src_v)

        def issue_gathers(c, b):
            pltpu.async_copy(qh_h.at[dst_v.at[c]], qv.at[b], sem_g[b])
            pltpu.async_copy(kh_h.at[src_v.at[c]], kv.at[b], sem_g[b])

        def wait_gathers(c, b):
            pltpu.make_async_copy(qh_h.at[dst_v.at[c]], qv.at[b], sem_g[b]).wait()
            pltpu.make_async_copy(kh_h.at[src_v.at[c]], kv.at[b], sem_g[b]).wait()

        def issue_writes(c, b):
            base = pl.multiple_of(wid * _EPW + c * _C, _C)
            pltpu.async_copy(qv.at[b], qg_h.at[pl.ds(base, _C)], sem_w[b])
            pltpu.async_copy(kv.at[b], kg_h.at[pl.ds(base, _C)], sem_w[b])

        def wait_writes(c, b):
            base = pl.multiple_of(wid * _EPW + c * _C, _C)
            pltpu.make_async_copy(qv.at[b], qg_h.at[pl.ds(base, _C)], sem_w[b]).wait()
            pltpu.make_async_copy(kv.at[b], kg_h.at[pl.ds(base, _C)], sem_w[b]).wait()

        issue_gathers(0, 0)
        wait_gathers(0, 0)
        issue_writes(0, 0)
        issue_gathers(1, 1)
        wait_gathers(1, 1)
        issue_writes(1, 1)
        wait_writes(0, 0)
        issue_gathers(2, 0)

        def body(j, carry):
            for k_ in range(2):
                c = 2 * j + k_
                b = k_
                wait_gathers(c, b)
                issue_writes(c, b)
                wait_writes(c - 1, 1 - b)
                issue_gathers(c + 1, 1 - b)
            return carry

        lax.fori_loop(1, _NCHUNK // 2, body, 0)
        c_last = _NCHUNK - 1
        wait_gathers(c_last, 0)
        issue_writes(c_last, 0)
        wait_writes(c_last - 1, 1)
        wait_writes(c_last, 0)

    return k(qh, kh, dst3, src3)


def _sc_v_segsum(vh, dst3, src3, zeros):
    mesh = plsc.VectorSubcoreMesh(core_axis_name="c", subcore_axis_name="s")

    @functools.partial(
        pl.kernel,
        out_type=jax.ShapeDtypeStruct((_NC, N_NODES, HIDDEN), jnp.float32),
        mesh=mesh,
        scratch_types=(
            pltpu.VMEM((_NCHUNK, _C), jnp.int32),
            pltpu.VMEM((2, _C), jnp.int32),
            pltpu.VMEM((2, _C, HIDDEN), jnp.float32),
            pltpu.VMEM_SHARED((N_NODES, HIDDEN), jnp.float32),
            pltpu.SemaphoreType.DMA,
            pltpu.SemaphoreType.DMA,
            pltpu.SemaphoreType.DMA,
            pltpu.SemaphoreType.DMA,
        ),
    )
    def k(vh_h, dst_h, src_h, zeros_h, aggp_h,
          dst_v, src_v, vv, agg_sh, sem_g0, sem_g1, sem_i0, sem_i1):
        cid = lax.axis_index("c")
        sid = lax.axis_index("s")
        wid = sid * _NC + cid
        sem_g = (sem_g0, sem_g1)
        sem_i = (sem_i0, sem_i1)

        _init_shared(zeros_h, agg_sh, sid)
        pltpu.sync_copy(dst_h.at[wid], dst_v)
        plsc.subcore_barrier()

        def issue_idx(c, b):
            pltpu.async_copy(src_h.at[wid, c], src_v.at[b], sem_i[b])

        def wait_idx(c, b):
            pltpu.make_async_copy(src_h.at[wid, c], src_v.at[b], sem_i[b]).wait()

        def issue_gather(b):
            pltpu.async_copy(vh_h.at[src_v.at[b]], vv.at[b], sem_g[b])

        def wait_gather(b):
            pltpu.make_async_copy(vh_h.at[src_v.at[b]], vv.at[b], sem_g[b]).wait()

        issue_idx(0, 0)
        wait_idx(0, 0)
        issue_gather(0)
        issue_idx(1, 1)

        def body(j, carry):
            for k_ in range(2):
                c = 2 * j + k_
                b = k_
                wait_gather(b)
                wait_idx(c + 1, 1 - b)
                issue_gather(1 - b)

                @pl.when(c + 2 < _NCHUNK)
                def _():
                    issue_idx(c + 2, b)

                pltpu.sync_copy(vv.at[b], agg_sh.at[dst_v.at[c]], add=True)
            return carry

        lax.fori_loop(0, (_NCHUNK - 1) // 2, body, 0)
        c_last = _NCHUNK - 1
        wait_gather(0)
        pltpu.sync_copy(vv.at[0], agg_sh.at[dst_v.at[c_last]], add=True)

        plsc.subcore_barrier()
        _copy_out_partial(agg_sh, aggp_h, cid, sid)

    return k(vh, dst3, src3, zeros)


def _sc_segsum(conn, dst3, zeros):
    mesh = plsc.VectorSubcoreMesh(core_axis_name="c", subcore_axis_name="s")

    @functools.partial(
        pl.kernel,
        out_type=jax.ShapeDtypeStruct((_NC, N_NODES, HIDDEN), jnp.float32),
        mesh=mesh,
        scratch_types=(
            pltpu.VMEM((_NCHUNK, _C), jnp.int32),
            pltpu.VMEM((2, _C, HIDDEN), jnp.float32),
            pltpu.VMEM_SHARED((N_NODES, HIDDEN), jnp.float32),
            pltpu.SemaphoreType.DMA,
            pltpu.SemaphoreType.DMA,
        ),
    )
    def k(conn_h, dst_h, zeros_h, eaggp_h, dst_v, cv, e_sh, sem_l0, sem_l1):
        cid = lax.axis_index("c")
        sid = lax.axis_index("s")
        wid = sid * _NC + cid
        sem_l = (sem_l0, sem_l1)

        _init_shared(zeros_h, e_sh, sid)
        pltpu.sync_copy(dst_h.at[wid], dst_v)
        plsc.subcore_barrier()

        def issue_load(c, b):
            base = pl.multiple_of(wid * _EPW + c * _C, _C)
            pltpu.async_copy(conn_h.at[pl.ds(base, _C)], cv.at[b], sem_l[b])

        def wait_load(c, b):
            base = pl.multiple_of(wid * _EPW + c * _C, _C)
            pltpu.make_async_copy(conn_h.at[pl.ds(base, _C)], cv.at[b],
                                  sem_l[b]).wait()

        issue_load(0, 0)

        def body(j, carry):
            for k_ in range(2):
                c = 2 * j + k_
                b = k_
                wait_load(c, b)
                issue_load(c + 1, 1 - b)
                pltpu.sync_copy(cv.at[b], e_sh.at[dst_v.at[c]], add=True)
            return carry

        lax.fori_loop(0, (_NCHUNK - 1) // 2, body, 0)
        c_last = _NCHUNK - 1
        wait_load(c_last, 0)
        pltpu.sync_copy(cv.at[0], e_sh.at[dst_v.at[c_last]], add=True)

        plsc.subcore_barrier()
        _copy_out_partial(e_sh, eaggp_h, cid, sid)

    return k(conn, dst3, zeros)


def _tc_qkv(x, qkv_weight, qkv_bias):
    bm = 1000

    def body(x_ref, w_ref, b_ref, q_ref, k_ref, v_ref):
        r = lax.dot_general(
            x_ref[...], w_ref[...], (((1,), (1,)), ((), ())),
            preferred_element_type=jnp.float32,
            precision=lax.Precision.HIGHEST,
        ) + b_ref[...]
        q_ref[...] = r[:, :A]
        k_ref[...] = r[:, A:2 * A]
        v_ref[...] = r[:, 2 * A:]

    return pl.pallas_call(
        body,
        grid=(N_NODES // bm,),
        in_specs=[
            pl.BlockSpec((bm, HIDDEN), lambda i: (i, 0)),
            pl.BlockSpec((3 * A, HIDDEN), lambda i: (0, 0)),
            pl.BlockSpec((1, 3 * A), lambda i: (0, 0)),
        ],
        out_specs=[pl.BlockSpec((bm, HIDDEN), lambda i: (i, 0))] * 3,
        out_shape=[jax.ShapeDtypeStruct((N_NODES, HIDDEN), jnp.float32)] * 3,
    )(x, qkv_weight, qkv_bias)


def _tc_edge(poly_conn, qg, kg, E_weight, E_bias):
    be = 2000

    def body(pc_ref, qg_ref, kg_ref, w_ref, b_ref, conn_ref):
        eh = lax.dot_general(
            pc_ref[...], w_ref[...], (((1,), (1,)), ((), ())),
            preferred_element_type=jnp.float32,
            precision=lax.Precision.HIGHEST,
        ) + b_ref[...]
        m = qg_ref[...] + kg_ref[...]
        c1 = m * eh[:, :A]
        c2 = jnp.sqrt(jnp.maximum(c1, 0.0)) - jnp.sqrt(jnp.maximum(-c1, 0.0))
        conn_ref[...] = jnp.maximum(c2 + eh[:, A:], 0.0)

    return pl.pallas_call(
        body,
        grid=(N_EDGES // be,),
        in_specs=[
            pl.BlockSpec((be, HIDDEN), lambda i: (i, 0)),
            pl.BlockSpec((be, HIDDEN), lambda i: (i, 0)),
            pl.BlockSpec((be, HIDDEN), lambda i: (i, 0)),
            pl.BlockSpec((2 * A, HIDDEN), lambda i: (0, 0)),
            pl.BlockSpec((1, 2 * A), lambda i: (0, 0)),
        ],
        out_specs=pl.BlockSpec((be, HIDDEN), lambda i: (i, 0)),
        out_shape=jax.ShapeDtypeStruct((N_EDGES, HIDDEN), jnp.float32),
    )(poly_conn, qg, kg, E_weight, E_bias)


def _tc_final(a0, a1, e0, e1, w, b):
    bm = 1000

    def body(a0_ref, a1_ref, e0_ref, e1_ref, w_ref, b_ref, o_ref):
        eagg = e0_ref[...] + e1_ref[...]
        o_ref[...] = a0_ref[...] + a1_ref[...] + lax.dot_general(
            eagg, w_ref[...], (((1,), (1,)), ((), ())),
            preferred_element_type=jnp.float32,
            precision=lax.Precision.HIGHEST,
        ) + b_ref[...]

    blk = pl.BlockSpec((bm, HIDDEN), lambda i: (i, 0))
    return pl.pallas_call(
        body,
        grid=(N_NODES // bm,),
        in_specs=[blk, blk, blk, blk,
                  pl.BlockSpec((HIDDEN, A), lambda i: (0, 0)),
                  pl.BlockSpec((1, HIDDEN), lambda i: (0, 0))],
        out_specs=blk,
        out_shape=jax.ShapeDtypeStruct((N_NODES, HIDDEN), jnp.float32),
    )(a0, a1, e0, e1, w, b)


def kernel(x, poly_conn, poly_index, qkv_weight, qkv_bias, E_weight, E_bias,
           conn_lin_weight, conn_lin_bias):
    qh, kh, vh = _tc_qkv(x, qkv_weight, qkv_bias.reshape(1, -1))
    dst3 = poly_index[0].reshape(_NW, _NCHUNK, _C)
    src3 = poly_index[1].reshape(_NW, _NCHUNK, _C)
    zeros = jnp.zeros((N_NODES, HIDDEN), jnp.float32)
    qg, kg = _sc_qk_gather(qh, kh, dst3, src3)
    aggp = _sc_v_segsum(vh, dst3, src3, zeros)
    conn = _tc_edge(poly_conn, qg, kg, E_weight, E_bias.reshape(1, -1))
    eaggp = _sc_segsum(conn, dst3, zeros)
    no = _tc_final(aggp[0], aggp[1], eaggp[0], eaggp[1],
                   conn_lin_weight, conn_lin_bias.reshape(1, -1))
    return no, conn


# speedup vs baseline: 6.0104x; 1.1180x over previous
import functools

import jax
import jax.numpy as jnp
from jax import lax
from jax.experimental import pallas as pl
from jax.experimental.pallas import tpu as pltpu
from jax.experimental.pallas import tpu_sc as plsc

N_NODES = 10000
N_EDGES = 320000
HIDDEN = 128
A = 128

_INFO = plsc.get_sparse_core_info()
_NC = _INFO.num_cores
_NS = _INFO.num_subcores
_NW = _NC * _NS
_EPW = N_EDGES // _NW
_C = 80
_NCHUNK = _EPW // _C
_RPT = 624
_REM_BASE = _NS * _RPT


def _init_shared(zeros_h, sh, sid):
    base = pl.multiple_of(sid * _RPT, 8)
    pltpu.sync_copy(zeros_h.at[pl.ds(base, _RPT)], sh.at[pl.ds(base, _RPT)])

    @pl.when(sid < 2)
    def _():
        off = pl.multiple_of(_REM_BASE + sid * 8, 8)
        pltpu.sync_copy(zeros_h.at[pl.ds(off, 8)], sh.at[pl.ds(off, 8)])


def _copy_out_partial(sh, out_h, cid, sid):
    base = pl.multiple_of(sid * _RPT, 8)
    pltpu.sync_copy(sh.at[pl.ds(base, _RPT)], out_h.at[cid, pl.ds(base, _RPT)])

    @pl.when(sid < 2)
    def _():
        off = pl.multiple_of(_REM_BASE + sid * 8, 8)
        pltpu.sync_copy(sh.at[pl.ds(off, 8)], out_h.at[cid, pl.ds(off, 8)])


def _sc_qk_gather(qh, kh, dst3, src3):
    mesh = plsc.VectorSubcoreMesh(core_axis_name="c", subcore_axis_name="s")

    @functools.partial(
        pl.kernel,
        out_type=(
            jax.ShapeDtypeStruct((N_EDGES, HIDDEN), jnp.float32),
            jax.ShapeDtypeStruct((N_EDGES, HIDDEN), jnp.float32),
        ),
        mesh=mesh,
        scratch_types=(
            pltpu.VMEM((_NCHUNK, _C), jnp.int32),
            pltpu.VMEM((_NCHUNK, _C), jnp.int32),
            pltpu.VMEM((2, _C, HIDDEN), jnp.float32),
            pltpu.VMEM((2, _C, HIDDEN), jnp.float32),
            pltpu.SemaphoreType.DMA,
            pltpu.SemaphoreType.DMA,
            pltpu.SemaphoreType.DMA,
            pltpu.SemaphoreType.DMA,
        ),
    )
    def k(qh_h, kh_h, dst_h, src_h, qg_h, kg_h,
          dst_v, src_v, qv, kv, sem_g0, sem_g1, sem_w0, sem_w1):
        cid = lax.axis_index("c")
        sid = lax.axis_index("s")
        wid = sid * _NC + cid
        sem_g = (sem_g0, sem_g1)
        sem_w = (sem_w0, sem_w1)

        pltpu.sync_copy(dst_h.at[wid], dst_v)
        pltpu.sync_copy(src_h.at[wid], src_v)

        def issue_gathers(c, b):
            pltpu.async_copy(qh_h.at[dst_v.at[c]], qv.at[b], sem_g[b])
            pltpu.async_copy(kh_h.at[src_v.at[c]], kv.at[b], sem_g[b])

        def wait_gathers(c, b):
            pltpu.make_async_copy(qh_h.at[dst_v.at[c]], qv.at[b], sem_g[b]).wait()
            pltpu.make_async_copy(kh_h.at[src_v.at[c]], kv.at[b], sem_g[b]).wait()

        def issue_writes(c, b):
            base = pl.multiple_of(wid * _EPW + c * _C, _C)
            pltpu.async_copy(qv.at[b], qg_h.at[pl.ds(base, _C)], sem_w[b])
            pltpu.async_copy(kv.at[b], kg_h.at[pl.ds(base, _C)], sem_w[b])

        def wait_writes(c, b):
            base = pl.multiple_of(wid * _EPW + c * _C, _C)
            pltpu.make_async_copy(qv.at[b], qg_h.at[pl.ds(base, _C)], sem_w[b]).wait()
            pltpu.make_async_copy(kv.at[b], kg_h.at[pl.ds(base, _C)], sem_w[b]).wait()

        issue_gathers(0, 0)
        wait_gathers(0, 0)
        issue_writes(0, 0)
        issue_gathers(1, 1)
        wait_gathers(1, 1)
        issue_writes(1, 1)
        wait_writes(0, 0)
        issue_gathers(2, 0)

        def body(j, carry):
            for k_ in range(2):
                c = 2 * j + k_
                b = k_
                wait_gathers(c, b)
                issue_writes(c, b)
                wait_writes(c - 1, 1 - b)
                issue_gathers(c + 1, 1 - b)
            return carry

        lax.fori_loop(1, _NCHUNK // 2, body, 0)
        c_last = _NCHUNK - 1
        wait_gathers(c_last, 0)
        issue_writes(c_last, 0)
        wait_writes(c_last - 1, 1)
        wait_writes(c_last, 0)

    return k(qh, kh, dst3, src3)


def _sc_v_segsum(vh, dst3, src3, zeros):
    mesh = plsc.VectorSubcoreMesh(core_axis_name="c", subcore_axis_name="s")

    @functools.partial(
        pl.kernel,
        out_type=jax.ShapeDtypeStruct((_NC, N_NODES, HIDDEN), jnp.float32),
        mesh=mesh,
        scratch_types=(
            pltpu.VMEM((_NCHUNK, _C), jnp.int32),
            pltpu.VMEM((2, _C), jnp.int32),
            pltpu.VMEM((2, _C, HIDDEN), jnp.float32),
            pltpu.VMEM_SHARED((N_NODES, HIDDEN), jnp.float32),
            pltpu.SemaphoreType.DMA,
            pltpu.SemaphoreType.DMA,
            pltpu.SemaphoreType.DMA,
            pltpu.SemaphoreType.DMA,
        ),
    )
    def k(vh_h, dst_h, src_h, zeros_h, aggp_h,
          dst_v, src_v, vv, agg_sh, sem_g0, sem_g1, sem_i0, sem_i1):
        cid = lax.axis_index("c")
        sid = lax.axis_index("s")
        wid = sid * _NC + cid
        sem_g = (sem_g0, sem_g1)
        sem_i = (sem_i0, sem_i1)

        _init_shared(zeros_h, agg_sh, sid)
        pltpu.sync_copy(dst_h.at[wid], dst_v)
        plsc.subcore_barrier()

        def issue_idx(c, b):
            pltpu.async_copy(src_h.at[wid, c], src_v.at[b], sem_i[b])

        def wait_idx(c, b):
            pltpu.make_async_copy(src_h.at[wid, c], src_v.at[b], sem_i[b]).wait()

        def issue_gather(b):
            pltpu.async_copy(vh_h.at[src_v.at[b]], vv.at[b], sem_g[b])

        def wait_gather(b):
            pltpu.make_async_copy(vh_h.at[src_v.at[b]], vv.at[b], sem_g[b]).wait()

        issue_idx(0, 0)
        wait_idx(0, 0)
        issue_gather(0)
        issue_idx(1, 1)

        def body(j, carry):
            for k_ in range(2):
                c = 2 * j + k_
                b = k_
                wait_gather(b)
                wait_idx(c + 1, 1 - b)
                issue_gather(1 - b)

                @pl.when(c + 2 < _NCHUNK)
                def _():
                    issue_idx(c + 2, b)

                pltpu.sync_copy(vv.at[b], agg_sh.at[dst_v.at[c]], add=True)
            return carry

        lax.fori_loop(0, (_NCHUNK - 1) // 2, body, 0)
        c_last = _NCHUNK - 1
        wait_gather(0)
        pltpu.sync_copy(vv.at[0], agg_sh.at[dst_v.at[c_last]], add=True)

        plsc.subcore_barrier()
        _copy_out_partial(agg_sh, aggp_h, cid, sid)

    return k(vh, dst3, src3, zeros)


def _sc_segsum(conn, dst3, zeros):
    mesh = plsc.VectorSubcoreMesh(core_axis_name="c", subcore_axis_name="s")

    @functools.partial(
        pl.kernel,
        out_type=jax.ShapeDtypeStruct((_NC, N_NODES, HIDDEN), jnp.float32),
        mesh=mesh,
        scratch_types=(
            pltpu.VMEM((_NCHUNK, _C), jnp.int32),
            pltpu.VMEM((2, _C, HIDDEN), jnp.float32),
            pltpu.VMEM_SHARED((N_NODES, HIDDEN), jnp.float32),
            pltpu.SemaphoreType.DMA,
            pltpu.SemaphoreType.DMA,
        ),
    )
    def k(conn_h, dst_h, zeros_h, eaggp_h, dst_v, cv, e_sh, sem_l0, sem_l1):
        cid = lax.axis_index("c")
        sid = lax.axis_index("s")
        wid = sid * _NC + cid
        sem_l = (sem_l0, sem_l1)

        _init_shared(zeros_h, e_sh, sid)
        pltpu.sync_copy(dst_h.at[wid], dst_v)
        plsc.subcore_barrier()

        def issue_load(c, b):
            base = pl.multiple_of(wid * _EPW + c * _C, _C)
            pltpu.async_copy(conn_h.at[pl.ds(base, _C)], cv.at[b], sem_l[b])

        def wait_load(c, b):
            base = pl.multiple_of(wid * _EPW + c * _C, _C)
            pltpu.make_async_copy(conn_h.at[pl.ds(base, _C)], cv.at[b],
                                  sem_l[b]).wait()

        issue_load(0, 0)

        def body(j, carry):
            for k_ in range(2):
                c = 2 * j + k_
                b = k_
                wait_load(c, b)
                issue_load(c + 1, 1 - b)
                pltpu.sync_copy(cv.at[b], e_sh.at[dst_v.at[c]], add=True)
            return carry

        lax.fori_loop(0, (_NCHUNK - 1) // 2, body, 0)
        c_last = _NCHUNK - 1
        wait_load(c_last, 0)
        pltpu.sync_copy(cv.at[0], e_sh.at[dst_v.at[c_last]], add=True)

        plsc.subcore_barrier()
        _copy_out_partial(e_sh, eaggp_h, cid, sid)

    return k(conn, dst3, zeros)


def _tc_qkv(x, qkv_weight, qkv_bias):
    bm = 1000

    def body(x_ref, w_ref, b_ref, q_ref, k_ref, v_ref):
        r = lax.dot_general(
            x_ref[...], w_ref[...], (((1,), (1,)), ((), ())),
            preferred_element_type=jnp.float32,
            precision=lax.Precision.HIGHEST,
        ) + b_ref[...]
        q_ref[...] = r[:, :A]
        k_ref[...] = r[:, A:2 * A]
        v_ref[...] = r[:, 2 * A:]

    return pl.pallas_call(
        body,
        grid=(N_NODES // bm,),
        in_specs=[
            pl.BlockSpec((bm, HIDDEN), lambda i: (i, 0)),
            pl.BlockSpec((3 * A, HIDDEN), lambda i: (0, 0)),
            pl.BlockSpec((1, 3 * A), lambda i: (0, 0)),
        ],
        out_specs=[pl.BlockSpec((bm, HIDDEN), lambda i: (i, 0))] * 3,
        out_shape=[jax.ShapeDtypeStruct((N_NODES, HIDDEN), jnp.float32)] * 3,
    )(x, qkv_weight, qkv_bias)


def _tc_edge(poly_conn, qg, kg, E_weight, E_bias):
    be = 2000

    def body(pc_ref, qg_ref, kg_ref, w_ref, b_ref, conn_ref):
        eh = lax.dot_general(
            pc_ref[...], w_ref[...], (((1,), (1,)), ((), ())),
            preferred_element_type=jnp.float32,
            precision=lax.Precision.DEFAULT,
        ) + b_ref[...]
        m = qg_ref[...] + kg_ref[...]
        c1 = m * eh[:, :A]
        c2 = jnp.sqrt(jnp.maximum(c1, 0.0)) - jnp.sqrt(jnp.maximum(-c1, 0.0))
        conn_ref[...] = jnp.maximum(c2 + eh[:, A:], 0.0)

    return pl.pallas_call(
        body,
        grid=(N_EDGES // be,),
        in_specs=[
            pl.BlockSpec((be, HIDDEN), lambda i: (i, 0)),
            pl.BlockSpec((be, HIDDEN), lambda i: (i, 0)),
            pl.BlockSpec((be, HIDDEN), lambda i: (i, 0)),
            pl.BlockSpec((2 * A, HIDDEN), lambda i: (0, 0)),
            pl.BlockSpec((1, 2 * A), lambda i: (0, 0)),
        ],
        out_specs=pl.BlockSpec((be, HIDDEN), lambda i: (i, 0)),
        out_shape=jax.ShapeDtypeStruct((N_EDGES, HIDDEN), jnp.float32),
    )(poly_conn, qg, kg, E_weight, E_bias)


def _tc_final(a0, a1, e0, e1, w, b):
    bm = 1000

    def body(a0_ref, a1_ref, e0_ref, e1_ref, w_ref, b_ref, o_ref):
        eagg = e0_ref[...] + e1_ref[...]
        o_ref[...] = a0_ref[...] + a1_ref[...] + lax.dot_general(
            eagg, w_ref[...], (((1,), (1,)), ((), ())),
            preferred_element_type=jnp.float32,
            precision=lax.Precision.HIGHEST,
        ) + b_ref[...]

    blk = pl.BlockSpec((bm, HIDDEN), lambda i: (i, 0))
    return pl.pallas_call(
        body,
        grid=(N_NODES // bm,),
        in_specs=[blk, blk, blk, blk,
                  pl.BlockSpec((HIDDEN, A), lambda i: (0, 0)),
                  pl.BlockSpec((1, HIDDEN), lambda i: (0, 0))],
        out_specs=blk,
        out_shape=jax.ShapeDtypeStruct((N_NODES, HIDDEN), jnp.float32),
    )(a0, a1, e0, e1, w, b)


def kernel(x, poly_conn, poly_index, qkv_weight, qkv_bias, E_weight, E_bias,
           conn_lin_weight, conn_lin_bias):
    qh, kh, vh = _tc_qkv(x, qkv_weight, qkv_bias.reshape(1, -1))
    dst3 = poly_index[0].reshape(_NW, _NCHUNK, _C)
    src3 = poly_index[1].reshape(_NW, _NCHUNK, _C)
    zeros = jnp.zeros((N_NODES, HIDDEN), jnp.float32)
    qg, kg = _sc_qk_gather(qh, kh, dst3, src3)
    aggp = _sc_v_segsum(vh, dst3, src3, zeros)
    conn = _tc_edge(poly_conn, qg, kg, E_weight, E_bias.reshape(1, -1))
    eaggp = _sc_segsum(conn, dst3, zeros)
    no = _tc_final(aggp[0], aggp[1], eaggp[0], eaggp[1],
                   conn_lin_weight, conn_lin_bias.reshape(1, -1))
    return no, conn
